# baseline (speedup 1.0000x reference)
import functools

import jax
import jax.numpy as jnp
from jax import lax
from jax.experimental import pallas as pl
from jax.experimental.pallas import tpu as pltpu
from jax.experimental.pallas import tpu_sc as plsc

DAY = 288
D = 128
B_TOTAL = 4096 * 200
NW = 32
B_PER_W = B_TOTAL // NW
G = 64
NG = 1
W = G * NG
NGRP = B_PER_W // W
NBUF = 10
L = 16


def _make_sc_call():
    mesh = plsc.VectorSubcoreMesh(core_axis_name="c", subcore_axis_name="s")

    @functools.partial(
        pl.kernel,
        out_type=jax.ShapeDtypeStruct((B_TOTAL, D), jnp.float32),
        mesh=mesh,
        scratch_types=(
            [pltpu.VMEM_SHARED((DAY, D), jnp.float32)]
            + [pltpu.VMEM((B_PER_W,), jnp.float32)]
            + [pltpu.VMEM((NBUF * NG, G), jnp.int32)]
            + [pltpu.VMEM((W, D), jnp.float32) for _ in range(NBUF)]
            + [pltpu.SemaphoreType.DMA for _ in range(2 * NBUF)]
        ),
    )
    def sc_embed(x_hbm, table_hbm, out_hbm, table_sp, x_v, idx_v, *rest):
        rows = rest[:NBUF]
        gsem = rest[NBUF:2 * NBUF]
        wsem = rest[2 * NBUF:]

        wid = lax.axis_index("s") * 2 + lax.axis_index("c")
        base = wid * B_PER_W

        sid = lax.axis_index("s")

        @pl.when(sid < DAY // 32)
        def _():
            off = pl.multiple_of(sid * 32, 32)
            pltpu.sync_copy(table_hbm.at[pl.ds(off, 32)],
                            table_sp.at[pl.ds(off, 32)])

        pltpu.sync_copy(x_hbm.at[pl.ds(base, B_PER_W)], x_v)
        plsc.subcore_barrier()

        def compute_idx(g, b):
            for h in range(NG):
                for i in range(G // L):
                    xv = x_v[pl.ds(g * W + h * G + i * L, L)]
                    idx_v[NG * b + h, pl.ds(i * L, L)] = (
                        xv * float(DAY)).astype(jnp.int32)

        def gathers(b):
            return [
                pltpu.make_async_copy(
                    table_sp.at[idx_v.at[NG * b + h]],
                    rows[b].at[pl.ds(h * G, G)], gsem[b])
                for h in range(NG)
            ]

        def write(b, g):
            return pltpu.make_async_copy(
                rows[b], out_hbm.at[pl.ds(base + g * W, W)], wsem[b])

        for b in range(NBUF):
            compute_idx(b, b)
            for c in gathers(b):
                c.start()

        def body(go, _):
            for b in range(NBUF):
                gg = go * NBUF + b
                for c in gathers(b):
                    c.wait()
                w = write(b, gg)
                w.start()
                compute_idx(gg + NBUF, b)
                w.wait()
                for c in gathers(b):
                    c.start()
            return _

        lax.fori_loop(0, (NGRP - NBUF) // NBUF, body, None)

        for b in range(NBUF):
            for c in gathers(b):
                c.wait()
            write(b, NGRP - NBUF + b).start()
        for b in range(NBUF):
            write(b, NGRP - NBUF + b).wait()

    return sc_embed


_sc_embed = _make_sc_call()


@jax.jit
def kernel(x, day_embed):
    out = _sc_embed(x.reshape(B_TOTAL), day_embed)
    return out.reshape(x.shape[0], x.shape[1], D)


# speedup vs baseline: 1.0016x; 1.0016x over previous
import functools

import jax
import jax.numpy as jnp
from jax import lax
from jax.experimental import pallas as pl
from jax.experimental.pallas import tpu as pltpu
from jax.experimental.pallas import tpu_sc as plsc

DAY = 288
D = 128
B_TOTAL = 4096 * 200
NW = 32
B_PER_W = B_TOTAL // NW
G = 64
NG = 1
W = G * NG
NGRP = B_PER_W // W
NBUF = 8
L = 16


def _make_sc_call():
    mesh = plsc.VectorSubcoreMesh(core_axis_name="c", subcore_axis_name="s")

    @functools.partial(
        pl.kernel,
        out_type=jax.ShapeDtypeStruct((B_TOTAL, D), jnp.float32),
        mesh=mesh,
        scratch_types=(
            [pltpu.VMEM_SHARED((DAY, D), jnp.float32)]
            + [pltpu.VMEM((B_PER_W,), jnp.float32)]
            + [pltpu.VMEM((NBUF * NG, G), jnp.int32)]
            + [pltpu.VMEM((W, D), jnp.float32) for _ in range(NBUF)]
            + [pltpu.SemaphoreType.DMA for _ in range(2 * NBUF)]
        ),
    )
    def sc_embed(x_hbm, table_hbm, out_hbm, table_sp, x_v, idx_v, *rest):
        rows = rest[:NBUF]
        gsem = rest[NBUF:2 * NBUF]
        wsem = rest[2 * NBUF:]

        wid = lax.axis_index("s") * 2 + lax.axis_index("c")
        base = wid * B_PER_W

        sid = lax.axis_index("s")

        @pl.when(sid < DAY // 32)
        def _():
            off = pl.multiple_of(sid * 32, 32)
            pltpu.sync_copy(table_hbm.at[pl.ds(off, 32)],
                            table_sp.at[pl.ds(off, 32)])

        pltpu.sync_copy(x_hbm.at[pl.ds(base, B_PER_W)], x_v)
        plsc.subcore_barrier()

        def compute_idx(g, b):
            for h in range(NG):
                for i in range(G // L):
                    xv = x_v[pl.ds(g * W + h * G + i * L, L)]
                    idx_v[NG * b + h, pl.ds(i * L, L)] = (
                        xv * float(DAY)).astype(jnp.int32)

        def gathers(b):
            return [
                pltpu.make_async_copy(
                    table_sp.at[idx_v.at[NG * b + h]],
                    rows[b].at[pl.ds(h * G, G)], gsem[b])
                for h in range(NG)
            ]

        def write(b, g):
            return pltpu.make_async_copy(
                rows[b], out_hbm.at[pl.ds(base + g * W, W)], wsem[b])

        for b in range(NBUF):
            compute_idx(b, b)
            for c in gathers(b):
                c.start()

        def body(go, _):
            for b in range(NBUF):
                gg = go * NBUF + b
                for c in gathers(b):
                    c.wait()
                w = write(b, gg)
                w.start()
                compute_idx(gg + NBUF, b)
                w.wait()
                for c in gathers(b):
                    c.start()
            return _

        lax.fori_loop(0, (NGRP - NBUF) // NBUF, body, None)

        for b in range(NBUF):
            for c in gathers(b):
                c.wait()
            write(b, NGRP - NBUF + b).start()
        for b in range(NBUF):
            write(b, NGRP - NBUF + b).wait()

    return sc_embed


_sc_embed = _make_sc_call()


@jax.jit
def kernel(x, day_embed):
    out = _sc_embed(x.reshape(B_TOTAL), day_embed)
    return out.reshape(x.shape[0], x.shape[1], D)
